# Initial kernel scaffold; baseline (speedup 1.0000x reference)
import jax
import jax.numpy as jnp
from jax.experimental import pallas as pl


def kernel(logits):
    raise NotImplementedError("write your pallas kernel here")


# speedup vs baseline: 35.1489x; 35.1489x over previous
import jax
import jax.numpy as jnp
from jax.experimental import pallas as pl
from jax.experimental.pallas import tpu as pltpu

T = 1.3
KREG = 5
LAMDA = 0.01
TAU = 0.93
TOPK = 98

NPAD = 100096
R = 16


def _softmax_topk_kernel(x_ref, scores_ref, topkz_ref, topki_ref, s_ref, w_ref):
    y = x_ref[...] / T
    m = jnp.max(y, axis=1, keepdims=True)
    z = jnp.exp(y - m)
    s = jnp.sum(z, axis=1, keepdims=True)
    scores_ref[...] = z / s
    s_ref[...] = s
    w_ref[...] = z
    colid = jax.lax.broadcasted_iota(jnp.int32, (R, NPAD), 1)
    lane = jax.lax.broadcasted_iota(jnp.int32, (R, 128), 1)

    def body(i, carry):
        accz, acci = carry
        w = w_ref[...]
        mx = jnp.max(w, axis=1, keepdims=True)
        fi = jnp.min(jnp.where(w == mx, colid, NPAD), axis=1, keepdims=True)
        w_ref[...] = jnp.where(colid == fi, -1.0, w)
        return jnp.where(lane == i, mx, accz), jnp.where(lane == i, fi, acci)

    accz, acci = jax.lax.fori_loop(
        0, TOPK, body,
        (jnp.zeros((R, 128), jnp.float32), jnp.full((R, 128), NPAD, jnp.int32)),
    )
    topkz_ref[...] = accz
    topki_ref[...] = acci


def kernel(logits):
    b, n = logits.shape
    xpad = jnp.pad(logits, ((0, 0), (0, NPAD - n)), constant_values=-jnp.inf)

    scores_p, topkz, topki, s = pl.pallas_call(
        _softmax_topk_kernel,
        grid=(b // R,),
        in_specs=[pl.BlockSpec((R, NPAD), lambda i: (i, 0))],
        out_specs=[
            pl.BlockSpec((R, NPAD), lambda i: (i, 0)),
            pl.BlockSpec((R, 128), lambda i: (i, 0)),
            pl.BlockSpec((R, 128), lambda i: (i, 0)),
            pl.BlockSpec((R, 1), lambda i: (i, 0)),
        ],
        out_shape=[
            jax.ShapeDtypeStruct((b, NPAD), jnp.float32),
            jax.ShapeDtypeStruct((b, 128), jnp.float32),
            jax.ShapeDtypeStruct((b, 128), jnp.int32),
            jax.ShapeDtypeStruct((b, 1), jnp.float32),
        ],
        scratch_shapes=[pltpu.VMEM((R, NPAD), jnp.float32)],
        compiler_params=pltpu.CompilerParams(
            dimension_semantics=("parallel",)
        ),
    )(xpad)

    ordered = topkz / s
    cums = jnp.cumsum(ordered, axis=1)
    pen = LAMDA * (jnp.arange(128) >= KREG).astype(jnp.float32)
    pcs = jnp.cumsum(pen)
    sizes_base = jnp.sum((cums + pcs[None, :] <= TAU).astype(jnp.int32), axis=1) + 1
    idx = sizes_base - 1
    ord_at = jnp.take_along_axis(ordered, idx[:, None], axis=1)[:, 0]
    cum_at = jnp.take_along_axis(cums, idx[:, None], axis=1)[:, 0]
    pcs_at = pcs[idx]
    V = (TAU - (cum_at - ord_at) - pcs_at) / jnp.maximum(ord_at, 1e-30)
    U = jax.random.uniform(jax.random.key(1), sizes_base.shape)
    sizes = sizes_base - (U >= V).astype(jnp.int32)
    sizes = jnp.maximum(sizes, 0)

    member = jnp.arange(128)[None, :] < sizes[:, None]
    mask = jnp.zeros((b, n), jnp.bool_).at[
        jnp.arange(b)[:, None], topki
    ].set(member, mode="drop")

    return scores_p[:, :n], sizes, mask


# speedup vs baseline: 42.7769x; 1.2170x over previous
import jax
import jax.numpy as jnp
from jax.experimental import pallas as pl
from jax.experimental.pallas import tpu as pltpu

T = 1.3
KREG = 5
LAMDA = 0.01
TAU = 0.93
ROUNDS = 99

NPAD = 100096
NCHUNK = 782
R = 16


def _topk_kernel(x_ref, topv_ref, m_ref, s_ref, z_ref):
    y = x_ref[...] / T
    m = jnp.max(y, axis=1, keepdims=True)
    z = jnp.exp(y - m)
    s = jnp.sum(z, axis=1, keepdims=True)
    z_ref[...] = z
    m_ref[...] = m
    s_ref[...] = s
    lane = jax.lax.broadcasted_iota(jnp.int32, (R, 128), 1)

    def body(j, carry):
        vprev, pos, acc = carry
        w = z_ref[...]
        cprev = jnp.sum(
            jnp.where(w == vprev, 1, 0).astype(jnp.int32), axis=1, keepdims=True
        )
        vj = jnp.max(jnp.where(w < vprev, w, -1.0), axis=1, keepdims=True)
        acc = jnp.where((lane >= pos) & (lane < pos + cprev), vprev, acc)
        return vj, pos + cprev, acc

    _, _, acc = jax.lax.fori_loop(
        0, ROUNDS, body,
        (jnp.full((R, 1), jnp.inf, jnp.float32),
         jnp.zeros((R, 1), jnp.int32),
         jnp.zeros((R, 128), jnp.float32)),
    )
    topv_ref[...] = acc


def _scores_mask_kernel(x_ref, m_ref, s_ref, vstar_ref, cstar_ref,
                        scores_ref, mask_ref):
    z = jnp.exp(x_ref[...] / T - m_ref[...])
    scores_ref[...] = z / s_ref[...]
    vstar = vstar_ref[...]
    z3 = z.reshape(R, NCHUNK, 128)
    t3 = (z3 == vstar[:, :, None]).astype(jnp.float32)
    scnt = jnp.sum(t3, axis=2)
    mc = (jax.lax.broadcasted_iota(jnp.int32, (NCHUNK, NCHUNK), 0)
          < jax.lax.broadcasted_iota(jnp.int32, (NCHUNK, NCHUNK), 1)
          ).astype(jnp.float32)
    pchunk = jnp.dot(scnt, mc, preferred_element_type=jnp.float32)
    ml = (jax.lax.broadcasted_iota(jnp.int32, (128, 128), 0)
          < jax.lax.broadcasted_iota(jnp.int32, (128, 128), 1)
          ).astype(jnp.float32)
    wlane = jnp.dot(t3.reshape(R * NCHUNK, 128), ml,
                    preferred_element_type=jnp.float32).reshape(R, NCHUNK, 128)
    tie_rank = (pchunk[:, :, None] + wlane).reshape(R, NPAD)
    cstar = cstar_ref[...]
    mask_ref[...] = (z > vstar) | ((z == vstar) & (tie_rank < cstar))


def kernel(logits):
    b, n = logits.shape
    xpad = jnp.pad(logits, ((0, 0), (0, NPAD - n)), constant_values=-jnp.inf)

    topv, m, s = pl.pallas_call(
        _topk_kernel,
        grid=(b // R,),
        in_specs=[pl.BlockSpec((R, NPAD), lambda i: (i, 0))],
        out_specs=[
            pl.BlockSpec((R, 128), lambda i: (i, 0)),
            pl.BlockSpec((R, 1), lambda i: (i, 0)),
            pl.BlockSpec((R, 1), lambda i: (i, 0)),
        ],
        out_shape=[
            jax.ShapeDtypeStruct((b, 128), jnp.float32),
            jax.ShapeDtypeStruct((b, 1), jnp.float32),
            jax.ShapeDtypeStruct((b, 1), jnp.float32),
        ],
        scratch_shapes=[pltpu.VMEM((R, NPAD), jnp.float32)],
        compiler_params=pltpu.CompilerParams(
            dimension_semantics=("parallel",)
        ),
    )(xpad)

    ordered = topv / s
    cums = jnp.cumsum(ordered, axis=1)
    pen = LAMDA * (jnp.arange(128) >= KREG).astype(jnp.float32)
    pcs = jnp.cumsum(pen)
    sizes_base = jnp.sum((cums + pcs[None, :] <= TAU).astype(jnp.int32), axis=1) + 1
    idx = sizes_base - 1
    ord_at = jnp.take_along_axis(ordered, idx[:, None], axis=1)[:, 0]
    cum_at = jnp.take_along_axis(cums, idx[:, None], axis=1)[:, 0]
    pcs_at = pcs[idx]
    V = (TAU - (cum_at - ord_at) - pcs_at) / jnp.maximum(ord_at, 1e-30)
    U = jax.random.uniform(jax.random.key(1), sizes_base.shape)
    sizes = sizes_base - (U >= V).astype(jnp.int32)
    sizes = jnp.maximum(sizes, 0)

    thr_idx = jnp.maximum(sizes - 1, 0)
    vstar = jnp.take_along_axis(topv, thr_idx[:, None], axis=1)
    vstar = jnp.where(sizes[:, None] == 0, jnp.inf, vstar)
    gt = jnp.sum((topv > vstar).astype(jnp.int32), axis=1, keepdims=True)
    cstar = (sizes[:, None] - gt).astype(jnp.float32)

    scores_p, mask_p = pl.pallas_call(
        _scores_mask_kernel,
        grid=(b // R,),
        in_specs=[
            pl.BlockSpec((R, NPAD), lambda i: (i, 0)),
            pl.BlockSpec((R, 1), lambda i: (i, 0)),
            pl.BlockSpec((R, 1), lambda i: (i, 0)),
            pl.BlockSpec((R, 1), lambda i: (i, 0)),
            pl.BlockSpec((R, 1), lambda i: (i, 0)),
        ],
        out_specs=[
            pl.BlockSpec((R, NPAD), lambda i: (i, 0)),
            pl.BlockSpec((R, NPAD), lambda i: (i, 0)),
        ],
        out_shape=[
            jax.ShapeDtypeStruct((b, NPAD), jnp.float32),
            jax.ShapeDtypeStruct((b, NPAD), jnp.bool_),
        ],
        compiler_params=pltpu.CompilerParams(
            dimension_semantics=("parallel",)
        ),
    )(xpad, m, s, vstar, cstar)

    return scores_p[:, :n], sizes, mask_p[:, :n]


# speedup vs baseline: 155.3426x; 3.6315x over previous
import functools

import jax
import jax.numpy as jnp
from jax import lax
from jax.experimental import pallas as pl
from jax.experimental.pallas import tpu as pltpu
from jax.experimental.pallas import tpu_sc as plsc

T = 1.3
KREG = 5
LAMDA = 0.01
TAU = 0.93

NPAD = 100096
NCHUNK = 782
R = 16
NV = 6256
NG1 = 391
NG1P = 400
NG2 = 25
TOPK = 98


def _softmax_z_kernel(x_ref, z_out_ref, m_ref, s_ref):
    y = x_ref[...] / T
    m = jnp.max(y, axis=1, keepdims=True)
    z = jnp.exp(y - m)
    s = jnp.sum(z, axis=1, keepdims=True)
    z_out_ref[...] = z
    m_ref[...] = m
    s_ref[...] = s


def _sc_topk_kernel(z_hbm, out_hbm, zv, g1, g2, tvs):
    wid = lax.axis_index("s") * 2 + lax.axis_index("c")
    lane16 = lax.iota(jnp.int32, 16)

    def vec(ref, i):
        return ref[pl.ds(i * 16, 16)]

    for rr in range(4):
        row = wid * 4 + rr
        pltpu.sync_copy(z_hbm.at[row], zv)

        def g1_body(g, _):
            acc = vec(zv, g * 16)
            for i in range(1, 16):
                acc = jnp.maximum(acc, vec(zv, g * 16 + i))
            g1[pl.ds(g * 16, 16)] = acc
            return 0

        lax.fori_loop(0, NG1, g1_body, 0, unroll=False)
        for g in range(NG1, NG1P):
            g1[pl.ds(g * 16, 16)] = jnp.full((16,), -1.0, jnp.float32)

        def g2_body(h, _):
            acc = vec(g1, h * 16)
            for i in range(1, 16):
                acc = jnp.maximum(acc, vec(g1, h * 16 + i))
            g2[pl.ds(h * 16, 16)] = acc
            return 0

        lax.fori_loop(0, NG2, g2_body, 0, unroll=False)

        def round_body(j, tv):
            def tmax(h, acc):
                return jnp.maximum(acc, vec(g2, h))

            t = lax.fori_loop(0, NG2, tmax,
                              jnp.full((16,), -2.0, jnp.float32))
            gmax = jnp.max(t)

            def fh(h, best):
                hit = jnp.max(jnp.where(vec(g2, h) == gmax, 1, 0)) > 0
                return jnp.where((best == NG2) & hit, h, best)

            hstar = lax.fori_loop(0, NG2, fh, NG2)

            def fg(i, best):
                hit = jnp.max(
                    jnp.where(vec(g1, hstar * 16 + i) == gmax, 1, 0)) > 0
                return jnp.where((best == 16) & hit, i, best)

            gstar = hstar * 16 + lax.fori_loop(0, 16, fg, 16)

            def fv(i, best):
                hit = jnp.max(
                    jnp.where(vec(zv, gstar * 16 + i) == gmax, 1, 0)) > 0
                return jnp.where((best == 16) & hit, i, best)

            vstar = gstar * 16 + lax.fori_loop(0, 16, fv, 16)

            v = vec(zv, vstar)
            eq = v == gmax
            lidx = plsc.all_reduce_ffs(eq)
            zv[pl.ds(vstar * 16, 16)] = jnp.where(
                eq & (lane16 == lidx), -1.0, v)

            acc = vec(zv, gstar * 16)
            for i in range(1, 16):
                acc = jnp.maximum(acc, vec(zv, gstar * 16 + i))
            g1[pl.ds(gstar * 16, 16)] = acc
            acc2 = vec(g1, hstar * 16)
            for i in range(1, 16):
                acc2 = jnp.maximum(acc2, vec(g1, hstar * 16 + i))
            g2[pl.ds(hstar * 16, 16)] = acc2

            jq, jr = j // 16, j % 16
            return tuple(
                jnp.where((jq == k) & (lane16 == jr), gmax, tv[k])
                for k in range(8)
            )

        tv = lax.fori_loop(
            0, TOPK, round_body,
            tuple(jnp.zeros((16,), jnp.float32) for _ in range(8)),
        )
        for k in range(8):
            tvs[pl.ds(k * 16, 16)] = tv[k]
        pltpu.sync_copy(tvs, out_hbm.at[row])


def _scores_mask_kernel(x_ref, m_ref, s_ref, vstar_ref, cstar_ref,
                        scores_ref, mask_ref):
    z = jnp.exp(x_ref[...] / T - m_ref[...])
    scores_ref[...] = z / s_ref[...]
    vstar = vstar_ref[...]
    z3 = z.reshape(R, NCHUNK, 128)
    t3 = (z3 == vstar[:, :, None]).astype(jnp.float32)
    scnt = jnp.sum(t3, axis=2)
    mc = (jax.lax.broadcasted_iota(jnp.int32, (NCHUNK, NCHUNK), 0)
          < jax.lax.broadcasted_iota(jnp.int32, (NCHUNK, NCHUNK), 1)
          ).astype(jnp.float32)
    pchunk = jnp.dot(scnt, mc, preferred_element_type=jnp.float32)
    ml = (jax.lax.broadcasted_iota(jnp.int32, (128, 128), 0)
          < jax.lax.broadcasted_iota(jnp.int32, (128, 128), 1)
          ).astype(jnp.float32)
    wlane = jnp.dot(t3.reshape(R * NCHUNK, 128), ml,
                    preferred_element_type=jnp.float32).reshape(R, NCHUNK, 128)
    tie_rank = (pchunk[:, :, None] + wlane).reshape(R, NPAD)
    cstar = cstar_ref[...]
    mask_ref[...] = (z > vstar) | ((z == vstar) & (tie_rank < cstar))


def kernel(logits):
    b, n = logits.shape
    xpad = jnp.pad(logits, ((0, 0), (0, NPAD - n)), constant_values=-jnp.inf)

    z, m, s = pl.pallas_call(
        _softmax_z_kernel,
        grid=(b // R,),
        in_specs=[pl.BlockSpec((R, NPAD), lambda i: (i, 0))],
        out_specs=[
            pl.BlockSpec((R, NPAD), lambda i: (i, 0)),
            pl.BlockSpec((R, 1), lambda i: (i, 0)),
            pl.BlockSpec((R, 1), lambda i: (i, 0)),
        ],
        out_shape=[
            jax.ShapeDtypeStruct((b, NPAD), jnp.float32),
            jax.ShapeDtypeStruct((b, 1), jnp.float32),
            jax.ShapeDtypeStruct((b, 1), jnp.float32),
        ],
        compiler_params=pltpu.CompilerParams(
            dimension_semantics=("parallel",)
        ),
    )(xpad)

    sc_topk = functools.partial(
        pl.kernel,
        out_type=jax.ShapeDtypeStruct((b, 128), jnp.float32),
        mesh=plsc.VectorSubcoreMesh(core_axis_name="c", subcore_axis_name="s"),
        compiler_params=pltpu.CompilerParams(needs_layout_passes=False),
        scratch_types=[
            pltpu.VMEM((NPAD,), jnp.float32),
            pltpu.VMEM((NG1P * 16,), jnp.float32),
            pltpu.VMEM((NG2 * 16,), jnp.float32),
            pltpu.VMEM((128,), jnp.float32),
        ],
    )(_sc_topk_kernel)
    topv = sc_topk(z)

    ordered = topv / s
    cums = jnp.cumsum(ordered, axis=1)
    pen = LAMDA * (jnp.arange(128) >= KREG).astype(jnp.float32)
    pcs = jnp.cumsum(pen)
    sizes_base = jnp.sum((cums + pcs[None, :] <= TAU).astype(jnp.int32), axis=1) + 1
    idx = sizes_base - 1
    ord_at = jnp.take_along_axis(ordered, idx[:, None], axis=1)[:, 0]
    cum_at = jnp.take_along_axis(cums, idx[:, None], axis=1)[:, 0]
    pcs_at = pcs[idx]
    V = (TAU - (cum_at - ord_at) - pcs_at) / jnp.maximum(ord_at, 1e-30)
    U = jax.random.uniform(jax.random.key(1), sizes_base.shape)
    sizes = sizes_base - (U >= V).astype(jnp.int32)
    sizes = jnp.maximum(sizes, 0)

    thr_idx = jnp.maximum(sizes - 1, 0)
    vstar = jnp.take_along_axis(topv, thr_idx[:, None], axis=1)
    vstar = jnp.where(sizes[:, None] == 0, jnp.inf, vstar)
    gt = jnp.sum((topv > vstar).astype(jnp.int32), axis=1, keepdims=True)
    cstar = (sizes[:, None] - gt).astype(jnp.float32)

    scores_p, mask_p = pl.pallas_call(
        _scores_mask_kernel,
        grid=(b // R,),
        in_specs=[
            pl.BlockSpec((R, NPAD), lambda i: (i, 0)),
            pl.BlockSpec((R, 1), lambda i: (i, 0)),
            pl.BlockSpec((R, 1), lambda i: (i, 0)),
            pl.BlockSpec((R, 1), lambda i: (i, 0)),
            pl.BlockSpec((R, 1), lambda i: (i, 0)),
        ],
        out_specs=[
            pl.BlockSpec((R, NPAD), lambda i: (i, 0)),
            pl.BlockSpec((R, NPAD), lambda i: (i, 0)),
        ],
        out_shape=[
            jax.ShapeDtypeStruct((b, NPAD), jnp.float32),
            jax.ShapeDtypeStruct((b, NPAD), jnp.bool_),
        ],
        compiler_params=pltpu.CompilerParams(
            dimension_semantics=("parallel",)
        ),
    )(xpad, m, s, vstar, cstar)

    return scores_p[:, :n], sizes, mask_p[:, :n]
